# baseline (speedup 1.0000x reference)
import functools
import jax
import jax.numpy as jnp
from jax import lax
from jax.experimental import pallas as pl
from jax.experimental.pallas import tpu as pltpu
from jax.experimental.pallas import tpu_sc as plsc

NC = 2
NS = 16
L = 16
CH = 40


def _vgather(x, idx):
    dn = lax.GatherDimensionNumbers(
        offset_dims=(), collapsed_slice_dims=(0,), start_index_map=(0,))
    return lax.gather(x, idx[:, None], dn, (1,),
                      mode=lax.GatherScatterMode.PROMISE_IN_BOUNDS)


def _make_edge_kernel(n, npad, e, hd2):
    nw = NC * NS
    assert e % (nw * CH) == 0
    ew_per = e // nw
    nch = ew_per // CH
    npairs = nch // 2
    assert nch % 2 == 0
    nvec = hd2 // L
    hv = 2
    nheads = nvec // hv
    wrows = CH
    assert n % wrows == 0
    nblk = n // wrows
    nblk_ceil = (nblk + NS - 1) // NS

    mesh = plsc.VectorSubcoreMesh(core_axis_name="c", subcore_axis_name="s")

    @functools.partial(
        pl.kernel,
        out_type=(jax.ShapeDtypeStruct((2 * npad, hd2), jnp.float32),
                  jax.ShapeDtypeStruct((2 * npad, L), jnp.float32)),
        mesh=mesh,
        compiler_params=pltpu.CompilerParams(use_tc_tiling_on_sc=False,
                                             needs_layout_passes=False),
        scratch_types=dict(
            acc_sh=pltpu.VMEM_SHARED((n, hd2), jnp.float32),
            accd_sh=pltpu.VMEM_SHARED((n, L), jnp.float32),
            ebuf0=pltpu.VMEM((4, CH), jnp.int32),
            ebuf1=pltpu.VMEM((4, CH), jnp.int32),
            sdst0=pltpu.VMEM((CH,), jnp.int32),
            sdst1=pltpu.VMEM((CH,), jnp.int32),
            xlr0=pltpu.VMEM((CH, hd2), jnp.float32),
            xlr1=pltpu.VMEM((CH, hd2), jnp.float32),
            xrr0=pltpu.VMEM((CH, hd2), jnp.float32),
            xrr1=pltpu.VMEM((CH, hd2), jnp.float32),
            stage0=pltpu.VMEM((CH, hd2), jnp.float32),
            stage1=pltpu.VMEM((CH, hd2), jnp.float32),
            staged0=pltpu.VMEM((CH, L), jnp.float32),
            staged1=pltpu.VMEM((CH, L), jnp.float32),
            wei=pltpu.VMEM((hd2,), jnp.float32),
            atti=pltpu.VMEM((hd2,), jnp.float32),
            gxl0=pltpu.SemaphoreType.DMA, gxl1=pltpu.SemaphoreType.DMA,
            gxr0=pltpu.SemaphoreType.DMA, gxr1=pltpu.SemaphoreType.DMA,
            isem0=pltpu.SemaphoreType.DMA, isem1=pltpu.SemaphoreType.DMA,
            ssem0=pltpu.SemaphoreType.DMA, ssem1=pltpu.SemaphoreType.DMA,
            ssn0=pltpu.SemaphoreType.DMA, ssn1=pltpu.SemaphoreType.DMA,
        ),
    )
    def edge_kernel(xl_hbm, xr_hbm, ep_hbm, we_hbm, att_hbm,
                    out_hbm, outd_hbm, acc_sh, accd_sh,
                    ebuf0, ebuf1, sdst0, sdst1, xlr0, xlr1, xrr0, xrr1,
                    stage0, stage1, staged0, staged1, wei, atti,
                    gxl0, gxl1, gxr0, gxr1, isem0, isem1,
                    ssem0, ssem1, ssn0, ssn1):
        cid = lax.axis_index("c")
        sid = lax.axis_index("s")
        wid = sid * NC + cid

        pltpu.sync_copy(we_hbm, wei)
        pltpu.sync_copy(att_hbm, atti)
        wv = [wei[pl.ds(i * L, L)] for i in range(nvec)]
        av = [atti[pl.ds(i * L, L)] for i in range(nvec)]
        lanes = lax.iota(jnp.int32, L)
        perms = [lanes ^ s for s in (8, 4, 2, 1)]
        hmask = [lanes == h for h in range(nheads)]

        zv = jnp.zeros((L,), jnp.float32)

        def zrow(r, _):
            for j in range(hd2 // L):
                stage0[r, pl.ds(j * L, L)] = zv
            staged0[r, pl.ds(0, L)] = zv
            return 0

        lax.fori_loop(0, CH, zrow, 0)

        def zblk(b, _):
            blk = b * NS + sid

            @pl.when(blk < nblk)
            def _():
                pltpu.sync_copy(stage0, acc_sh.at[pl.ds(blk * wrows, wrows)])
                pltpu.sync_copy(staged0,
                                accd_sh.at[pl.ds(blk * wrows, wrows)])

            return 0

        lax.fori_loop(0, nblk_ceil, zblk, 0)

        npadblk = (npad - n) // wrows
        @pl.when(sid < npadblk)
        def _():
            r = n + sid * wrows
            pltpu.sync_copy(stage0, out_hbm.at[pl.ds(cid * npad + r, wrows)])
            pltpu.sync_copy(staged0,
                            outd_hbm.at[pl.ds(cid * npad + r, wrows)])

        plsc.subcore_barrier()

        base0 = wid * ew_per
        par_refs = [
            (ebuf0, sdst0, xlr0, xrr0, stage0, staged0,
             gxl0, gxr0, isem0, ssem0, ssn0),
            (ebuf1, sdst1, xlr1, xrr1, stage1, staged1,
             gxl1, gxr1, isem1, ssem1, ssn1),
        ]

        for par in range(2):
            ebuf, _, xlr, xrr = par_refs[par][:4]
            gxl, gxr = par_refs[par][6:8]
            pltpu.sync_copy(ep_hbm.at[:, pl.ds(base0 + par * CH, CH)], ebuf)
            pltpu.async_copy(xl_hbm.at[ebuf.at[0]], xlr, gxl)
            pltpu.async_copy(xr_hbm.at[ebuf.at[1]], xrr, gxr)

        def compute_chunk(ebuf, xlr, xrr, stage, staged):
            def egroup(g, _):
                ewg = plsc.bitcast(ebuf[2, pl.ds(g * 8, L)], jnp.float32)
                ts = [g * 8 + i for i in range(8)]
                qs = []
                for i, t in enumerate(ts):
                    w = ewg[i]
                    qh = []
                    for h in range(nheads):
                        q = jnp.zeros((L,), jnp.float32)
                        for k in range(hv):
                            idx = h * hv + k
                            z = (xlr[t, pl.ds(idx * L, L)]
                                 + xrr[t, pl.ds(idx * L, L)]
                                 + w * wv[idx])
                            m = jnp.where(z >= 0.0, z, 0.2 * z)
                            q = q + m * av[idx]
                        qh.append(q)
                    qs.append(qh)
                for p in perms:
                    qs = [[q + _vgather(q, p) for q in qh] for qh in qs]
                for i, t in enumerate(ts):
                    den = jnp.zeros((L,), jnp.float32)
                    for h in range(nheads):
                        ex = jnp.exp(qs[i][h])
                        for k in range(hv):
                            idx = h * hv + k
                            stage[t, pl.ds(idx * L, L)] = (
                                xlr[t, pl.ds(idx * L, L)] * ex)
                        den = jnp.where(hmask[h], ex, den)
                    staged[t, pl.ds(0, L)] = den
                return 0

            lax.fori_loop(0, CH // 8, egroup, 0)

        def pair(j, _):
            for par in range(2):
                (ebuf, sdst, xlr, xrr, stage, staged,
                 gxl, gxr, isem, ssem, ssn) = par_refs[par]
                c = 2 * j + par
                base = base0 + c * CH
                pltpu.make_async_copy(xl_hbm.at[ebuf.at[0]], xlr, gxl).wait()
                pltpu.make_async_copy(xr_hbm.at[ebuf.at[1]], xrr, gxr).wait()

                @pl.when(j >= 1)
                def _():
                    pltpu.make_async_copy(stage, acc_sh.at[sdst], ssem).wait()
                    pltpu.make_async_copy(staged, accd_sh.at[sdst],
                                          ssn).wait()

                pltpu.async_copy(ep_hbm.at[1, pl.ds(base, CH)], sdst, isem)

                compute_chunk(ebuf, xlr, xrr, stage, staged)

                @pl.when(j < npairs - 1)
                def _():
                    pltpu.sync_copy(
                        ep_hbm.at[:, pl.ds(base + 2 * CH, CH)], ebuf)
                    pltpu.async_copy(xl_hbm.at[ebuf.at[0]], xlr, gxl)
                    pltpu.async_copy(xr_hbm.at[ebuf.at[1]], xrr, gxr)

                pltpu.make_async_copy(ep_hbm.at[1, pl.ds(base, CH)], sdst,
                                      isem).wait()
                pltpu.async_copy(stage, acc_sh.at[sdst], ssem, add=True)
                pltpu.async_copy(staged, accd_sh.at[sdst], ssn, add=True)
            return 0

        lax.fori_loop(0, npairs, pair, 0)
        for par in range(2):
            (_, sdst, _, _, stage, staged,
             _, _, _, ssem, ssn) = par_refs[par]
            pltpu.make_async_copy(stage, acc_sh.at[sdst], ssem).wait()
            pltpu.make_async_copy(staged, accd_sh.at[sdst], ssn).wait()
        plsc.subcore_barrier()

        def wblk(b, _):
            blk = b * NS + sid

            @pl.when(blk < nblk)
            def _():
                r = blk * wrows
                pltpu.sync_copy(acc_sh.at[pl.ds(r, wrows)],
                                out_hbm.at[pl.ds(cid * npad + r, wrows)])
                pltpu.sync_copy(accd_sh.at[pl.ds(r, wrows)],
                                outd_hbm.at[pl.ds(cid * npad + r, wrows)])

            return 0

        lax.fori_loop(0, nblk_ceil, wblk, 0)

    return edge_kernel


def _full(shape):
    return pl.BlockSpec(shape, lambda i: (0,) * len(shape))


def _make_prep_kernel(npad, t_len, d_in, hd2, blk):
    def body(x2, wl, bl, wr, br, xl, xr):
        xb = x2[:, 0:d_in]
        for t in range(1, t_len):
            xb = xb + x2[:, t * d_in:(t + 1) * d_in]
        xb = xb * (1.0 / t_len)
        xl[...] = jnp.dot(xb, wl[...],
                          preferred_element_type=jnp.float32) + bl[...]
        xr[...] = jnp.dot(xb, wr[...],
                          preferred_element_type=jnp.float32) + br[...]

    return pl.pallas_call(
        body,
        grid=(npad // blk,),
        in_specs=[
            pl.BlockSpec((blk, t_len * d_in), lambda i: (i, 0)),
            _full((d_in, hd2)), _full((1, hd2)),
            _full((d_in, hd2)), _full((1, hd2)),
        ],
        out_specs=[pl.BlockSpec((blk, hd2), lambda i: (i, 0))] * 2,
        out_shape=[jax.ShapeDtypeStruct((npad, hd2), jnp.float32)] * 2,
        compiler_params=pltpu.CompilerParams(
            dimension_semantics=("arbitrary",)),
    )


def _gat_epilogue(acc, accd, r_mat, m_mat, gb):
    a = acc[0] + acc[1]
    d = accd[0] + accd[1]
    den = jnp.dot(d, r_mat[...], preferred_element_type=jnp.float32)
    q = a / (den + 1e-16)
    return jax.nn.relu(
        jnp.dot(q, m_mat[...], preferred_element_type=jnp.float32) + gb[...])


def _make_mid_kernel(npad, hd2, hid, blk):
    def body(acc, accd, r_mat, m_mat, gb, wl, bl, wr, br, xl, xr):
        xp = _gat_epilogue(acc, accd, r_mat, m_mat, gb)
        xl[...] = jnp.dot(xp, wl[...],
                          preferred_element_type=jnp.float32) + bl[...]
        xr[...] = jnp.dot(xp, wr[...],
                          preferred_element_type=jnp.float32) + br[...]

    return pl.pallas_call(
        body,
        grid=(npad // blk,),
        in_specs=[
            pl.BlockSpec((2, blk, hd2), lambda i: (0, i, 0)),
            pl.BlockSpec((2, blk, L), lambda i: (0, i, 0)),
            _full((L, hd2)), _full((hd2, hid)), _full((1, hid)),
            _full((hid, hd2)), _full((1, hd2)),
            _full((hid, hd2)), _full((1, hd2)),
        ],
        out_specs=[pl.BlockSpec((blk, hd2), lambda i: (i, 0))] * 2,
        out_shape=[jax.ShapeDtypeStruct((npad, hd2), jnp.float32)] * 2,
        compiler_params=pltpu.CompilerParams(
            dimension_semantics=("arbitrary",)),
    )


def _make_fin_kernel(npad, t_len, d_in, hid, hd2, dk, heads, blk):
    nblk = npad // blk
    hd = dk // heads
    scale = hd ** -0.5
    tdi = t_len * d_in

    def body(acc, accd, r_mat, m_mat, gb, x2b, wpbig, bpbig,
             wqT, bq, wkT, bk, wvT, bv, woT, bo,
             fw1T, fb1, fw2T, fb2, rw1T, rb1, rw2T, rb2,
             tout, fout, rout, xs_s, q_s, k_s, v_s, srow_s):
        xp2 = _gat_epilogue(acc, accd, r_mat, m_mat, gb)
        xp = jnp.transpose(xp2)
        xsf = jnp.dot(wpbig[...], jnp.transpose(x2b[...]),
                      preferred_element_type=jnp.float32) + bpbig[...]
        xs_s[...] = xsf.reshape(t_len, hid, blk)

        def qkv(t, _):
            z = xs_s[t] + xp
            q_s[t] = jnp.dot(wqT[...], z,
                             preferred_element_type=jnp.float32) + bq[...]
            k_s[t] = jnp.dot(wkT[...], z,
                             preferred_element_type=jnp.float32) + bk[...]
            v_s[t] = jnp.dot(wvT[...], z,
                             preferred_element_type=jnp.float32) + bv[...]
            return 0

        lax.fori_loop(0, t_len, qkv, 0)

        def attend(t, _):
            qt = q_s[t] * scale

            def score(u, _):
                prod = (qt * k_s[u]).reshape(heads, hd, blk)
                srow_s[u] = jnp.sum(prod, axis=1)
                return 0

            lax.fori_loop(0, t_len, score, 0)
            s = srow_s[...]
            m = jnp.max(s, axis=0, keepdims=True)
            e = jnp.exp(s - m)
            srow_s[...] = e / jnp.sum(e, axis=0, keepdims=True)

            def accum(u, c):
                return c + srow_s[u][:, None, :] * v_s[u].reshape(
                    heads, hd, blk)

            ctx = lax.fori_loop(
                0, t_len, accum, jnp.zeros((heads, hd, blk), jnp.float32))
            ot = (jnp.dot(woT[...], ctx.reshape(dk, blk),
                          preferred_element_type=jnp.float32) + bo[...])
            tout[t] = jnp.transpose(ot)
            return 0

        lax.fori_loop(0, t_len, attend, 0)

        last = jnp.transpose(tout[t_len - 1])
        h1 = jax.nn.relu(jnp.dot(fw1T[...], last,
                                 preferred_element_type=jnp.float32)
                         + fb1[...])
        f = jax.nn.relu(jnp.dot(fw2T[...], h1,
                                preferred_element_type=jnp.float32)
                        + fb2[...])
        fout[...] = jnp.transpose(f)
        h2 = jax.nn.relu(jnp.dot(rw1T[...], last,
                                 preferred_element_type=jnp.float32)
                         + rb1[...])
        r = jax.nn.sigmoid(jnp.dot(rw2T[...], h2,
                                   preferred_element_type=jnp.float32)
                           + rb2[...])
        rout[...] = jnp.transpose(r)

    return pl.pallas_call(
        body,
        grid=(nblk,),
        in_specs=[
            pl.BlockSpec((2, blk, hd2), lambda i: (0, i, 0)),
            pl.BlockSpec((2, blk, L), lambda i: (0, i, 0)),
            _full((L, hd2)), _full((hd2, hid)), _full((1, hid)),
            pl.BlockSpec((blk, tdi), lambda i: (i, 0)),
            _full((t_len * hid, tdi)), _full((t_len * hid, 1)),
            _full((dk, hid)), _full((dk, 1)),
            _full((dk, hid)), _full((dk, 1)),
            _full((dk, hid)), _full((dk, 1)),
            _full((hid, dk)), _full((hid, 1)),
            _full((hid, hid)), _full((hid, 1)),
            _full((1, hid)), _full((1, 1)),
            _full((hid, hid)), _full((hid, 1)),
            _full((1, hid)), _full((1, 1)),
        ],
        out_specs=[
            pl.BlockSpec((t_len, blk, hid), lambda i: (0, i, 0)),
            pl.BlockSpec((blk, 1), lambda i: (i, 0)),
            pl.BlockSpec((blk, 1), lambda i: (i, 0)),
        ],
        out_shape=[
            jax.ShapeDtypeStruct((t_len, npad, hid), jnp.float32),
            jax.ShapeDtypeStruct((npad, 1), jnp.float32),
            jax.ShapeDtypeStruct((npad, 1), jnp.float32),
        ],
        scratch_shapes=[
            pltpu.VMEM((t_len, hid, blk), jnp.float32),
            pltpu.VMEM((t_len, dk, blk), jnp.float32),
            pltpu.VMEM((t_len, dk, blk), jnp.float32),
            pltpu.VMEM((t_len, dk, blk), jnp.float32),
            pltpu.VMEM((t_len, heads, blk), jnp.float32),
        ],
        compiler_params=pltpu.CompilerParams(
            dimension_semantics=("arbitrary",)),
    )


def kernel(x, edge_index, edge_weight, Wp, bp, g0_Wl, g0_bl, g0_Wr, g0_br,
           g0_We, g0_att, g0_b, g1_Wl, g1_bl, g1_Wr, g1_br, g1_We, g1_att,
           g1_b, Wq, bq, Wk, bk, Wv, bv, Wo, bo, f_W1, f_b1, f_W2, f_b2,
           r_W1, r_b1, r_W2, r_b2):
    n, t_len, d_in = x.shape
    e = edge_weight.shape[0]
    hid = Wp.shape[1]
    dk = Wq.shape[1]
    heads, c = g0_att.shape
    hd2 = heads * c
    blk = 400
    assert n % blk == 0
    src = edge_index[0]
    dst = edge_index[1]

    epack = jnp.concatenate(
        [src[None], dst[None],
         jax.lax.bitcast_convert_type(edge_weight, jnp.int32)[None],
         jnp.zeros((1, e), jnp.int32)], axis=0)
    x2 = x.reshape(n, t_len * d_in)
    r_mat = (jnp.arange(hd2)[None, :] // c == jnp.arange(L)[:, None]
             ).astype(jnp.float32)
    m_mat = (jnp.arange(hd2)[:, None] % c == jnp.arange(c)[None, :]
             ).astype(jnp.float32) / heads
    wpbig = jnp.kron(jnp.eye(t_len, dtype=jnp.float32), Wp.T)
    bpbig = jnp.tile(bp, t_len)[:, None]
    wl0 = Wp @ g0_Wl
    bl0 = (bp @ g0_Wl + g0_bl)[None]
    wr0 = Wp @ g0_Wr
    br0 = (bp @ g0_Wr + g0_br)[None]

    prep = _make_prep_kernel(n, t_len, d_in, hd2, 2000)
    xl0, xr0 = prep(x2, wl0, bl0, wr0, br0)

    ek = _make_edge_kernel(n, n, e, hd2)
    acc0, accd0 = ek(xl0, xr0, epack, g0_We.reshape(hd2),
                     g0_att.reshape(hd2))

    mid = _make_mid_kernel(n, hd2, hid, 2000)
    xl1, xr1 = mid(acc0.reshape(2, n, hd2), accd0.reshape(2, n, L),
                   r_mat, m_mat, g0_b[None],
                   g1_Wl, g1_bl[None], g1_Wr, g1_br[None])

    acc1, accd1 = ek(xl1, xr1, epack, g1_We.reshape(hd2),
                     g1_att.reshape(hd2))

    fin = _make_fin_kernel(n, t_len, d_in, hid, hd2, dk, heads, blk)
    tout, fout, rout = fin(
        acc1.reshape(2, n, hd2), accd1.reshape(2, n, L),
        r_mat, m_mat, g1_b[None], x2, wpbig, bpbig,
        Wq.T, bq[:, None], Wk.T, bk[:, None], Wv.T, bv[:, None],
        Wo.T, bo[:, None], f_W1.T, f_b1[:, None], f_W2.T, f_b2[:, None],
        r_W1.T, r_b1[:, None], r_W2.T, r_b2[:, None])
    return (fout, rout, tout)


# speedup vs baseline: 1.0362x; 1.0362x over previous
import functools
import jax
import jax.numpy as jnp
from jax import lax
from jax.experimental import pallas as pl
from jax.experimental.pallas import tpu as pltpu
from jax.experimental.pallas import tpu_sc as plsc

NC = 2
NS = 16
L = 16
CH = 40


def _vgather(x, idx):
    dn = lax.GatherDimensionNumbers(
        offset_dims=(), collapsed_slice_dims=(0,), start_index_map=(0,))
    return lax.gather(x, idx[:, None], dn, (1,),
                      mode=lax.GatherScatterMode.PROMISE_IN_BOUNDS)


def _make_edge_kernel(n, npad, e, hd2):
    nw = NC * NS
    assert e % (nw * CH) == 0
    ew_per = e // nw
    nch = ew_per // CH
    npairs = nch // 2
    assert nch % 2 == 0
    nvec = hd2 // L
    hv = 2
    nheads = nvec // hv
    wrows = CH
    assert n % wrows == 0
    nblk = n // wrows
    nblk_ceil = (nblk + NS - 1) // NS

    mesh = plsc.VectorSubcoreMesh(core_axis_name="c", subcore_axis_name="s")

    @functools.partial(
        pl.kernel,
        out_type=(jax.ShapeDtypeStruct((2 * npad, hd2), jnp.float32),
                  jax.ShapeDtypeStruct((2 * npad, L), jnp.float32)),
        mesh=mesh,
        compiler_params=pltpu.CompilerParams(use_tc_tiling_on_sc=False,
                                             needs_layout_passes=False),
        scratch_types=dict(
            acc_sh=pltpu.VMEM_SHARED((n, hd2), jnp.float32),
            accd_sh=pltpu.VMEM_SHARED((n, L), jnp.float32),
            ebuf0=pltpu.VMEM((4, CH), jnp.int32),
            ebuf1=pltpu.VMEM((4, CH), jnp.int32),
            sdst0=pltpu.VMEM((CH,), jnp.int32),
            sdst1=pltpu.VMEM((CH,), jnp.int32),
            xlr0=pltpu.VMEM((CH, hd2), jnp.float32),
            xlr1=pltpu.VMEM((CH, hd2), jnp.float32),
            xrr0=pltpu.VMEM((CH, hd2), jnp.float32),
            xrr1=pltpu.VMEM((CH, hd2), jnp.float32),
            stage0=pltpu.VMEM((CH, hd2), jnp.float32),
            stage1=pltpu.VMEM((CH, hd2), jnp.float32),
            staged0=pltpu.VMEM((CH, L), jnp.float32),
            staged1=pltpu.VMEM((CH, L), jnp.float32),
            wei=pltpu.VMEM((hd2,), jnp.float32),
            atti=pltpu.VMEM((hd2,), jnp.float32),
            gxl0=pltpu.SemaphoreType.DMA, gxl1=pltpu.SemaphoreType.DMA,
            gxr0=pltpu.SemaphoreType.DMA, gxr1=pltpu.SemaphoreType.DMA,
            isem0=pltpu.SemaphoreType.DMA, isem1=pltpu.SemaphoreType.DMA,
            ssem0=pltpu.SemaphoreType.DMA, ssem1=pltpu.SemaphoreType.DMA,
            ssn0=pltpu.SemaphoreType.DMA, ssn1=pltpu.SemaphoreType.DMA,
        ),
    )
    def edge_kernel(xl_hbm, xr_hbm, ep_hbm, we_hbm, att_hbm,
                    out_hbm, outd_hbm, acc_sh, accd_sh,
                    ebuf0, ebuf1, sdst0, sdst1, xlr0, xlr1, xrr0, xrr1,
                    stage0, stage1, staged0, staged1, wei, atti,
                    gxl0, gxl1, gxr0, gxr1, isem0, isem1,
                    ssem0, ssem1, ssn0, ssn1):
        cid = lax.axis_index("c")
        sid = lax.axis_index("s")
        wid = sid * NC + cid

        pltpu.sync_copy(we_hbm, wei)
        pltpu.sync_copy(att_hbm, atti)
        wv = [wei[pl.ds(i * L, L)] for i in range(nvec)]
        av = [atti[pl.ds(i * L, L)] for i in range(nvec)]
        lanes = lax.iota(jnp.int32, L)
        perms = [lanes ^ s for s in (8, 4, 2, 1)]
        hmask = [lanes == h for h in range(nheads)]

        zv = jnp.zeros((L,), jnp.float32)

        def zrow(r, _):
            for j in range(hd2 // L):
                stage0[r, pl.ds(j * L, L)] = zv
            staged0[r, pl.ds(0, L)] = zv
            return 0

        lax.fori_loop(0, CH, zrow, 0)

        def zblk(b, _):
            blk = b * NS + sid

            @pl.when(blk < nblk)
            def _():
                pltpu.sync_copy(stage0, acc_sh.at[pl.ds(blk * wrows, wrows)])
                pltpu.sync_copy(staged0,
                                accd_sh.at[pl.ds(blk * wrows, wrows)])

            return 0

        lax.fori_loop(0, nblk_ceil, zblk, 0)

        npadblk = (npad - n) // wrows
        @pl.when(sid < npadblk)
        def _():
            r = n + sid * wrows
            pltpu.sync_copy(stage0, out_hbm.at[pl.ds(cid * npad + r, wrows)])
            pltpu.sync_copy(staged0,
                            outd_hbm.at[pl.ds(cid * npad + r, wrows)])

        plsc.subcore_barrier()

        base0 = wid * ew_per
        par_refs = [
            (ebuf0, sdst0, xlr0, xrr0, stage0, staged0,
             gxl0, gxr0, isem0, ssem0, ssn0),
            (ebuf1, sdst1, xlr1, xrr1, stage1, staged1,
             gxl1, gxr1, isem1, ssem1, ssn1),
        ]

        for par in range(2):
            ebuf, _, xlr, xrr = par_refs[par][:4]
            gxl, gxr = par_refs[par][6:8]
            pltpu.sync_copy(ep_hbm.at[:, pl.ds(base0 + par * CH, CH)], ebuf)
            pltpu.async_copy(xl_hbm.at[ebuf.at[0]], xlr, gxl)
            pltpu.async_copy(xr_hbm.at[ebuf.at[1]], xrr, gxr)

        def compute_chunk(ebuf, xlr, xrr, stage, staged):
            def egroup(g, _):
                ewg = plsc.bitcast(ebuf[2, pl.ds(g * 8, L)], jnp.float32)
                ts = [g * 8 + i for i in range(8)]
                qs = []
                for i, t in enumerate(ts):
                    w = ewg[i]
                    qh = []
                    for h in range(nheads):
                        q = jnp.zeros((L,), jnp.float32)
                        for k in range(hv):
                            idx = h * hv + k
                            z = (xlr[t, pl.ds(idx * L, L)]
                                 + xrr[t, pl.ds(idx * L, L)]
                                 + w * wv[idx])
                            m = jnp.where(z >= 0.0, z, 0.2 * z)
                            q = q + m * av[idx]
                        qh.append(q)
                    qs.append(qh)
                for p in perms:
                    qs = [[q + _vgather(q, p) for q in qh] for qh in qs]
                for i, t in enumerate(ts):
                    den = jnp.zeros((L,), jnp.float32)
                    for h in range(nheads):
                        ex = jnp.exp(qs[i][h])
                        for k in range(hv):
                            idx = h * hv + k
                            stage[t, pl.ds(idx * L, L)] = (
                                xlr[t, pl.ds(idx * L, L)] * ex)
                        den = jnp.where(hmask[h], ex, den)
                    staged[t, pl.ds(0, L)] = den
                return 0

            lax.fori_loop(0, CH // 8, egroup, 0)

        def pair(j, _):
            for par in range(2):
                (ebuf, sdst, xlr, xrr, stage, staged,
                 gxl, gxr, isem, ssem, ssn) = par_refs[par]
                c = 2 * j + par
                base = base0 + c * CH
                pltpu.make_async_copy(xl_hbm.at[ebuf.at[0]], xlr, gxl).wait()
                pltpu.make_async_copy(xr_hbm.at[ebuf.at[1]], xrr, gxr).wait()

                @pl.when(j >= 1)
                def _():
                    pltpu.make_async_copy(stage, acc_sh.at[sdst], ssem).wait()
                    pltpu.make_async_copy(staged, accd_sh.at[sdst],
                                          ssn).wait()

                pltpu.async_copy(ep_hbm.at[1, pl.ds(base, CH)], sdst, isem)

                compute_chunk(ebuf, xlr, xrr, stage, staged)

                @pl.when(j < npairs - 1)
                def _():
                    pltpu.sync_copy(
                        ep_hbm.at[:, pl.ds(base + 2 * CH, CH)], ebuf)
                    pltpu.async_copy(xl_hbm.at[ebuf.at[0]], xlr, gxl)
                    pltpu.async_copy(xr_hbm.at[ebuf.at[1]], xrr, gxr)

                pltpu.make_async_copy(ep_hbm.at[1, pl.ds(base, CH)], sdst,
                                      isem).wait()
                pltpu.async_copy(stage, acc_sh.at[sdst], ssem, add=True)
                pltpu.async_copy(staged, accd_sh.at[sdst], ssn, add=True)
            return 0

        lax.fori_loop(0, npairs, pair, 0)
        for par in range(2):
            (_, sdst, _, _, stage, staged,
             _, _, _, ssem, ssn) = par_refs[par]
            pltpu.make_async_copy(stage, acc_sh.at[sdst], ssem).wait()
            pltpu.make_async_copy(staged, accd_sh.at[sdst], ssn).wait()
        plsc.subcore_barrier()

        def wblk(b, _):
            blk = b * NS + sid

            @pl.when(blk < nblk)
            def _():
                r = blk * wrows
                pltpu.sync_copy(acc_sh.at[pl.ds(r, wrows)],
                                out_hbm.at[pl.ds(cid * npad + r, wrows)])
                pltpu.sync_copy(accd_sh.at[pl.ds(r, wrows)],
                                outd_hbm.at[pl.ds(cid * npad + r, wrows)])

            return 0

        lax.fori_loop(0, nblk_ceil, wblk, 0)

    return edge_kernel


def _full(shape):
    return pl.BlockSpec(shape, lambda i: (0,) * len(shape))


def _make_prep_kernel(npad, t_len, d_in, hd2, blk):
    def body(x2, wl, bl, wr, br, xl, xr):
        xb = x2[:, 0:d_in]
        for t in range(1, t_len):
            xb = xb + x2[:, t * d_in:(t + 1) * d_in]
        xb = xb * (1.0 / t_len)
        xl[...] = jnp.dot(xb, wl[...],
                          preferred_element_type=jnp.float32) + bl[...]
        xr[...] = jnp.dot(xb, wr[...],
                          preferred_element_type=jnp.float32) + br[...]

    return pl.pallas_call(
        body,
        grid=(npad // blk,),
        in_specs=[
            pl.BlockSpec((blk, t_len * d_in), lambda i: (i, 0)),
            _full((d_in, hd2)), _full((1, hd2)),
            _full((d_in, hd2)), _full((1, hd2)),
        ],
        out_specs=[pl.BlockSpec((blk, hd2), lambda i: (i, 0))] * 2,
        out_shape=[jax.ShapeDtypeStruct((npad, hd2), jnp.float32)] * 2,
        compiler_params=pltpu.CompilerParams(
            dimension_semantics=("arbitrary",)),
    )


def _gat_epilogue(acc, accd, r_mat, m_mat, gb):
    a = acc[0] + acc[1]
    d = accd[0] + accd[1]
    den = jnp.dot(d, r_mat[...], preferred_element_type=jnp.float32)
    q = a / (den + 1e-16)
    return jax.nn.relu(
        jnp.dot(q, m_mat[...], preferred_element_type=jnp.float32) + gb[...])


def _make_mid_kernel(npad, hd2, hid, blk):
    def body(acc, accd, r_mat, m_mat, gb, wl, bl, wr, br, xl, xr):
        xp = _gat_epilogue(acc, accd, r_mat, m_mat, gb)
        xl[...] = jnp.dot(xp, wl[...],
                          preferred_element_type=jnp.float32) + bl[...]
        xr[...] = jnp.dot(xp, wr[...],
                          preferred_element_type=jnp.float32) + br[...]

    return pl.pallas_call(
        body,
        grid=(npad // blk,),
        in_specs=[
            pl.BlockSpec((2, blk, hd2), lambda i: (0, i, 0)),
            pl.BlockSpec((2, blk, L), lambda i: (0, i, 0)),
            _full((L, hd2)), _full((hd2, hid)), _full((1, hid)),
            _full((hid, hd2)), _full((1, hd2)),
            _full((hid, hd2)), _full((1, hd2)),
        ],
        out_specs=[pl.BlockSpec((blk, hd2), lambda i: (i, 0))] * 2,
        out_shape=[jax.ShapeDtypeStruct((npad, hd2), jnp.float32)] * 2,
        compiler_params=pltpu.CompilerParams(
            dimension_semantics=("arbitrary",)),
    )


def _make_fin_kernel(npad, t_len, d_in, hid, hd2, dk, heads, blk):
    nblk = npad // blk
    hd = dk // heads
    scale = hd ** -0.5
    tdi = t_len * d_in

    def body(acc, accd, r_mat, m_mat, gb, x2b, wpbig, bpbig,
             wqT, bq, wkT, bk, wvT, bv, woT, bo,
             fw1T, fb1, fw2T, fb2, rw1T, rb1, rw2T, rb2,
             tout, fout, rout, xs_s, q_s, k_s, v_s, srow_s):
        xp2 = _gat_epilogue(acc, accd, r_mat, m_mat, gb)
        xp = jnp.transpose(xp2)
        xsf = jnp.dot(wpbig[...], x2b[...],
                      preferred_element_type=jnp.float32) + bpbig[...]
        xs_s[...] = xsf.reshape(t_len, hid, blk)

        def qkv(t, _):
            z = xs_s[t] + xp
            q_s[t] = jnp.dot(wqT[...], z,
                             preferred_element_type=jnp.float32) + bq[...]
            k_s[t] = jnp.dot(wkT[...], z,
                             preferred_element_type=jnp.float32) + bk[...]
            v_s[t] = jnp.dot(wvT[...], z,
                             preferred_element_type=jnp.float32) + bv[...]
            return 0

        lax.fori_loop(0, t_len, qkv, 0)

        def attend(t, _):
            qt = q_s[t] * scale

            def score(u, _):
                prod = (qt * k_s[u]).reshape(heads, hd, blk)
                srow_s[u] = jnp.sum(prod, axis=1)
                return 0

            lax.fori_loop(0, t_len, score, 0)
            s = srow_s[...]
            m = jnp.max(s, axis=0, keepdims=True)
            e = jnp.exp(s - m)
            srow_s[...] = e / jnp.sum(e, axis=0, keepdims=True)

            def accum(u, c):
                return c + srow_s[u][:, None, :] * v_s[u].reshape(
                    heads, hd, blk)

            ctx = lax.fori_loop(
                0, t_len, accum, jnp.zeros((heads, hd, blk), jnp.float32))
            ot = (jnp.dot(woT[...], ctx.reshape(dk, blk),
                          preferred_element_type=jnp.float32) + bo[...])
            tout[t] = jnp.transpose(ot)
            return 0

        lax.fori_loop(0, t_len, attend, 0)

        last = jnp.transpose(tout[t_len - 1])
        h1 = jax.nn.relu(jnp.dot(fw1T[...], last,
                                 preferred_element_type=jnp.float32)
                         + fb1[...])
        f = jax.nn.relu(jnp.dot(fw2T[...], h1,
                                preferred_element_type=jnp.float32)
                        + fb2[...])
        fout[...] = jnp.transpose(f)
        h2 = jax.nn.relu(jnp.dot(rw1T[...], last,
                                 preferred_element_type=jnp.float32)
                         + rb1[...])
        r = jax.nn.sigmoid(jnp.dot(rw2T[...], h2,
                                   preferred_element_type=jnp.float32)
                           + rb2[...])
        rout[...] = jnp.transpose(r)

    return pl.pallas_call(
        body,
        grid=(nblk,),
        in_specs=[
            pl.BlockSpec((2, blk, hd2), lambda i: (0, i, 0)),
            pl.BlockSpec((2, blk, L), lambda i: (0, i, 0)),
            _full((L, hd2)), _full((hd2, hid)), _full((1, hid)),
            pl.BlockSpec((tdi, blk), lambda i: (0, i)),
            _full((t_len * hid, tdi)), _full((t_len * hid, 1)),
            _full((dk, hid)), _full((dk, 1)),
            _full((dk, hid)), _full((dk, 1)),
            _full((dk, hid)), _full((dk, 1)),
            _full((hid, dk)), _full((hid, 1)),
            _full((hid, hid)), _full((hid, 1)),
            _full((1, hid)), _full((1, 1)),
            _full((hid, hid)), _full((hid, 1)),
            _full((1, hid)), _full((1, 1)),
        ],
        out_specs=[
            pl.BlockSpec((t_len, blk, hid), lambda i: (0, i, 0)),
            pl.BlockSpec((blk, 1), lambda i: (i, 0)),
            pl.BlockSpec((blk, 1), lambda i: (i, 0)),
        ],
        out_shape=[
            jax.ShapeDtypeStruct((t_len, npad, hid), jnp.float32),
            jax.ShapeDtypeStruct((npad, 1), jnp.float32),
            jax.ShapeDtypeStruct((npad, 1), jnp.float32),
        ],
        scratch_shapes=[
            pltpu.VMEM((t_len, hid, blk), jnp.float32),
            pltpu.VMEM((t_len, dk, blk), jnp.float32),
            pltpu.VMEM((t_len, dk, blk), jnp.float32),
            pltpu.VMEM((t_len, dk, blk), jnp.float32),
            pltpu.VMEM((t_len, heads, blk), jnp.float32),
        ],
        compiler_params=pltpu.CompilerParams(
            dimension_semantics=("arbitrary",)),
    )


def kernel(x, edge_index, edge_weight, Wp, bp, g0_Wl, g0_bl, g0_Wr, g0_br,
           g0_We, g0_att, g0_b, g1_Wl, g1_bl, g1_Wr, g1_br, g1_We, g1_att,
           g1_b, Wq, bq, Wk, bk, Wv, bv, Wo, bo, f_W1, f_b1, f_W2, f_b2,
           r_W1, r_b1, r_W2, r_b2):
    n, t_len, d_in = x.shape
    e = edge_weight.shape[0]
    hid = Wp.shape[1]
    dk = Wq.shape[1]
    heads, c = g0_att.shape
    hd2 = heads * c
    blk = 512
    npad = -(-n // blk) * blk
    src = edge_index[0]
    dst = edge_index[1]

    epack = jnp.concatenate(
        [src[None], dst[None],
         jax.lax.bitcast_convert_type(edge_weight, jnp.int32)[None],
         jnp.zeros((1, e), jnp.int32)], axis=0)
    x2 = x.reshape(n, t_len * d_in)
    x2p = jnp.pad(x2, ((0, npad - n), (0, 0)))
    x2T = x2p.T
    r_mat = (jnp.arange(hd2)[None, :] // c == jnp.arange(L)[:, None]
             ).astype(jnp.float32)
    m_mat = (jnp.arange(hd2)[:, None] % c == jnp.arange(c)[None, :]
             ).astype(jnp.float32) / heads
    wpbig = jnp.kron(jnp.eye(t_len, dtype=jnp.float32), Wp.T)
    bpbig = jnp.tile(bp, t_len)[:, None]
    wl0 = Wp @ g0_Wl
    bl0 = (bp @ g0_Wl + g0_bl)[None]
    wr0 = Wp @ g0_Wr
    br0 = (bp @ g0_Wr + g0_br)[None]

    prep = _make_prep_kernel(npad, t_len, d_in, hd2, 2048)
    xl0, xr0 = prep(x2p, wl0, bl0, wr0, br0)

    ek = _make_edge_kernel(n, npad, e, hd2)
    acc0, accd0 = ek(xl0, xr0, epack, g0_We.reshape(hd2),
                     g0_att.reshape(hd2))

    mid = _make_mid_kernel(npad, hd2, hid, 2048)
    xl1, xr1 = mid(acc0.reshape(2, npad, hd2), accd0.reshape(2, npad, L),
                   r_mat, m_mat, g0_b[None],
                   g1_Wl, g1_bl[None], g1_Wr, g1_br[None])

    acc1, accd1 = ek(xl1, xr1, epack, g1_We.reshape(hd2),
                     g1_att.reshape(hd2))

    fin = _make_fin_kernel(npad, t_len, d_in, hid, hd2, dk, heads, blk)
    tout, fout, rout = fin(
        acc1.reshape(2, npad, hd2), accd1.reshape(2, npad, L),
        r_mat, m_mat, g1_b[None], x2T, wpbig, bpbig,
        Wq.T, bq[:, None], Wk.T, bk[:, None], Wv.T, bv[:, None],
        Wo.T, bo[:, None], f_W1.T, f_b1[:, None], f_W2.T, f_b2[:, None],
        r_W1.T, r_b1[:, None], r_W2.T, r_b2[:, None])
    return (fout[:n], rout[:n], tout[:, :n])


# speedup vs baseline: 1.0652x; 1.0280x over previous
import functools
import jax
import jax.numpy as jnp
from jax import lax
from jax.experimental import pallas as pl
from jax.experimental.pallas import tpu as pltpu
from jax.experimental.pallas import tpu_sc as plsc

NC = 2
NS = 16
L = 16
CH = 40


def _vgather(x, idx):
    dn = lax.GatherDimensionNumbers(
        offset_dims=(), collapsed_slice_dims=(0,), start_index_map=(0,))
    return lax.gather(x, idx[:, None], dn, (1,),
                      mode=lax.GatherScatterMode.PROMISE_IN_BOUNDS)


def _make_edge_kernel(n, npad, e, hd2):
    nw = NC * NS
    assert e % (nw * CH) == 0
    ew_per = e // nw
    nch = ew_per // CH
    npairs = nch // 2
    assert nch % 2 == 0
    nvec = hd2 // L
    hv = 2
    nheads = nvec // hv
    wrows = CH
    assert n % wrows == 0
    nblk = n // wrows
    nblk_ceil = (nblk + NS - 1) // NS

    mesh = plsc.VectorSubcoreMesh(core_axis_name="c", subcore_axis_name="s")

    @functools.partial(
        pl.kernel,
        out_type=(jax.ShapeDtypeStruct((2 * npad, hd2), jnp.float32),
                  jax.ShapeDtypeStruct((2 * npad, L), jnp.float32)),
        mesh=mesh,
        compiler_params=pltpu.CompilerParams(use_tc_tiling_on_sc=False,
                                             needs_layout_passes=False),
        scratch_types=dict(
            acc_sh=pltpu.VMEM_SHARED((n, hd2), jnp.float32),
            accd_sh=pltpu.VMEM_SHARED((n, L), jnp.float32),
            ebuf0=pltpu.VMEM((4, CH), jnp.int32),
            ebuf1=pltpu.VMEM((4, CH), jnp.int32),
            sdst0=pltpu.VMEM((CH,), jnp.int32),
            sdst1=pltpu.VMEM((CH,), jnp.int32),
            xlr0=pltpu.VMEM((CH, hd2), jnp.float32),
            xlr1=pltpu.VMEM((CH, hd2), jnp.float32),
            xrr0=pltpu.VMEM((CH, hd2), jnp.float32),
            xrr1=pltpu.VMEM((CH, hd2), jnp.float32),
            stage0=pltpu.VMEM((CH, hd2), jnp.float32),
            stage1=pltpu.VMEM((CH, hd2), jnp.float32),
            staged0=pltpu.VMEM((CH, L), jnp.float32),
            staged1=pltpu.VMEM((CH, L), jnp.float32),
            wei=pltpu.VMEM((hd2,), jnp.float32),
            atti=pltpu.VMEM((hd2,), jnp.float32),
            gxl0=pltpu.SemaphoreType.DMA, gxl1=pltpu.SemaphoreType.DMA,
            gxr0=pltpu.SemaphoreType.DMA, gxr1=pltpu.SemaphoreType.DMA,
            isem0=pltpu.SemaphoreType.DMA, isem1=pltpu.SemaphoreType.DMA,
            ssem0=pltpu.SemaphoreType.DMA, ssem1=pltpu.SemaphoreType.DMA,
            ssn0=pltpu.SemaphoreType.DMA, ssn1=pltpu.SemaphoreType.DMA,
        ),
    )
    def edge_kernel(xl_hbm, xr_hbm, ep_hbm, we_hbm, att_hbm,
                    out_hbm, outd_hbm, acc_sh, accd_sh,
                    ebuf0, ebuf1, sdst0, sdst1, xlr0, xlr1, xrr0, xrr1,
                    stage0, stage1, staged0, staged1, wei, atti,
                    gxl0, gxl1, gxr0, gxr1, isem0, isem1,
                    ssem0, ssem1, ssn0, ssn1):
        cid = lax.axis_index("c")
        sid = lax.axis_index("s")
        wid = sid * NC + cid

        pltpu.sync_copy(we_hbm, wei)
        pltpu.sync_copy(att_hbm, atti)
        wv = [wei[pl.ds(i * L, L)] for i in range(nvec)]
        av = [atti[pl.ds(i * L, L)] for i in range(nvec)]
        lanes = lax.iota(jnp.int32, L)
        perms = [lanes ^ s for s in (8, 4, 2, 1)]
        hmask = [lanes == h for h in range(nheads)]

        zv = jnp.zeros((L,), jnp.float32)

        def zrow(r, _):
            for j in range(hd2 // L):
                stage0[r, pl.ds(j * L, L)] = zv
            staged0[r, pl.ds(0, L)] = zv
            return 0

        lax.fori_loop(0, CH, zrow, 0)

        def zblk(b, _):
            blk = b * NS + sid

            @pl.when(blk < nblk)
            def _():
                pltpu.sync_copy(stage0, acc_sh.at[pl.ds(blk * wrows, wrows)])
                pltpu.sync_copy(staged0,
                                accd_sh.at[pl.ds(blk * wrows, wrows)])

            return 0

        lax.fori_loop(0, nblk_ceil, zblk, 0)

        npadblk = (npad - n) // wrows
        @pl.when(sid < npadblk)
        def _():
            r = n + sid * wrows
            pltpu.sync_copy(stage0, out_hbm.at[pl.ds(cid * npad + r, wrows)])
            pltpu.sync_copy(staged0,
                            outd_hbm.at[pl.ds(cid * npad + r, wrows)])

        plsc.subcore_barrier()

        base0 = wid * ew_per
        par_refs = [
            (ebuf0, sdst0, xlr0, xrr0, stage0, staged0,
             gxl0, gxr0, isem0, ssem0, ssn0),
            (ebuf1, sdst1, xlr1, xrr1, stage1, staged1,
             gxl1, gxr1, isem1, ssem1, ssn1),
        ]

        for par in range(2):
            ebuf, _, xlr, xrr = par_refs[par][:4]
            gxl, gxr = par_refs[par][6:8]
            pltpu.sync_copy(ep_hbm.at[:, pl.ds(base0 + par * CH, CH)], ebuf)
            pltpu.async_copy(xl_hbm.at[ebuf.at[0]], xlr, gxl)
            pltpu.async_copy(xr_hbm.at[ebuf.at[1]], xrr, gxr)

        def compute_chunk(ebuf, xlr, xrr, stage, staged):
            def egroup(g, _):
                ewg = plsc.bitcast(ebuf[2, pl.ds(g * 8, L)], jnp.float32)
                ts = [g * 8 + i for i in range(8)]
                qs = []
                for i, t in enumerate(ts):
                    w = ewg[i]
                    qh = []
                    for h in range(nheads):
                        q = jnp.zeros((L,), jnp.float32)
                        for k in range(hv):
                            idx = h * hv + k
                            z = (xlr[t, pl.ds(idx * L, L)]
                                 + xrr[t, pl.ds(idx * L, L)]
                                 + w * wv[idx])
                            m = jnp.where(z >= 0.0, z, 0.2 * z)
                            q = q + m * av[idx]
                        qh.append(q)
                    qs.append(qh)
                for p in perms:
                    qs = [[q + _vgather(q, p) for q in qh] for qh in qs]
                for i, t in enumerate(ts):
                    den = jnp.zeros((L,), jnp.float32)
                    for h in range(nheads):
                        ex = jnp.exp(qs[i][h])
                        for k in range(hv):
                            idx = h * hv + k
                            stage[t, pl.ds(idx * L, L)] = (
                                xlr[t, pl.ds(idx * L, L)] * ex)
                        den = jnp.where(hmask[h], ex, den)
                    staged[t, pl.ds(0, L)] = den
                return 0

            lax.fori_loop(0, CH // 8, egroup, 0)

        def pair(j, _):
            for par in range(2):
                (ebuf, sdst, xlr, xrr, stage, staged,
                 gxl, gxr, isem, ssem, ssn) = par_refs[par]
                c = 2 * j + par
                base = base0 + c * CH
                pltpu.make_async_copy(xl_hbm.at[ebuf.at[0]], xlr, gxl).wait()
                pltpu.make_async_copy(xr_hbm.at[ebuf.at[1]], xrr, gxr).wait()

                @pl.when(j >= 1)
                def _():
                    pltpu.make_async_copy(stage, acc_sh.at[sdst], ssem).wait()
                    pltpu.make_async_copy(staged, accd_sh.at[sdst],
                                          ssn).wait()

                pltpu.async_copy(ep_hbm.at[1, pl.ds(base, CH)], sdst, isem)

                compute_chunk(ebuf, xlr, xrr, stage, staged)

                @pl.when(j < npairs - 1)
                def _():
                    pltpu.sync_copy(
                        ep_hbm.at[:, pl.ds(base + 2 * CH, CH)], ebuf)
                    pltpu.async_copy(xl_hbm.at[ebuf.at[0]], xlr, gxl)
                    pltpu.async_copy(xr_hbm.at[ebuf.at[1]], xrr, gxr)

                pltpu.make_async_copy(ep_hbm.at[1, pl.ds(base, CH)], sdst,
                                      isem).wait()
                pltpu.async_copy(stage, acc_sh.at[sdst], ssem, add=True)
                pltpu.async_copy(staged, accd_sh.at[sdst], ssn, add=True)
            return 0

        lax.fori_loop(0, npairs, pair, 0)
        for par in range(2):
            (_, sdst, _, _, stage, staged,
             _, _, _, ssem, ssn) = par_refs[par]
            pltpu.make_async_copy(stage, acc_sh.at[sdst], ssem).wait()
            pltpu.make_async_copy(staged, accd_sh.at[sdst], ssn).wait()
        plsc.subcore_barrier()

        def wblk(b, _):
            blk = b * NS + sid

            @pl.when(blk < nblk)
            def _():
                r = blk * wrows
                pltpu.sync_copy(acc_sh.at[pl.ds(r, wrows)],
                                out_hbm.at[pl.ds(cid * npad + r, wrows)])
                pltpu.sync_copy(accd_sh.at[pl.ds(r, wrows)],
                                outd_hbm.at[pl.ds(cid * npad + r, wrows)])

            return 0

        lax.fori_loop(0, nblk_ceil, wblk, 0)

    return edge_kernel


def _full(shape):
    return pl.BlockSpec(shape, lambda i: (0,) * len(shape))


def _make_prep_kernel(npad, t_len, d_in, hd2, blk):
    def body(x2, wl, bl, wr, br, xl, xr):
        xb = x2[:, 0:d_in]
        for t in range(1, t_len):
            xb = xb + x2[:, t * d_in:(t + 1) * d_in]
        xb = xb * (1.0 / t_len)
        xl[...] = jnp.dot(xb, wl[...],
                          preferred_element_type=jnp.float32) + bl[...]
        xr[...] = jnp.dot(xb, wr[...],
                          preferred_element_type=jnp.float32) + br[...]

    return pl.pallas_call(
        body,
        grid=(npad // blk,),
        in_specs=[
            pl.BlockSpec((blk, t_len * d_in), lambda i: (i, 0)),
            _full((d_in, hd2)), _full((1, hd2)),
            _full((d_in, hd2)), _full((1, hd2)),
        ],
        out_specs=[pl.BlockSpec((blk, hd2), lambda i: (i, 0))] * 2,
        out_shape=[jax.ShapeDtypeStruct((npad, hd2), jnp.float32)] * 2,
        compiler_params=pltpu.CompilerParams(
            dimension_semantics=("arbitrary",)),
    )


def _gat_epilogue(acc, accd, r_mat, m_mat, gb):
    a = acc[0] + acc[1]
    d = accd[0] + accd[1]
    den = jnp.dot(d, r_mat[...], preferred_element_type=jnp.float32)
    q = a / (den + 1e-16)
    return jax.nn.relu(
        jnp.dot(q, m_mat[...], preferred_element_type=jnp.float32) + gb[...])


def _make_mid_kernel(npad, hd2, hid, blk):
    def body(acc, accd, r_mat, m_mat, gb, wl, bl, wr, br, xl, xr):
        xp = _gat_epilogue(acc, accd, r_mat, m_mat, gb)
        xl[...] = jnp.dot(xp, wl[...],
                          preferred_element_type=jnp.float32) + bl[...]
        xr[...] = jnp.dot(xp, wr[...],
                          preferred_element_type=jnp.float32) + br[...]

    return pl.pallas_call(
        body,
        grid=(npad // blk,),
        in_specs=[
            pl.BlockSpec((2, blk, hd2), lambda i: (0, i, 0)),
            pl.BlockSpec((2, blk, L), lambda i: (0, i, 0)),
            _full((L, hd2)), _full((hd2, hid)), _full((1, hid)),
            _full((hid, hd2)), _full((1, hd2)),
            _full((hid, hd2)), _full((1, hd2)),
        ],
        out_specs=[pl.BlockSpec((blk, hd2), lambda i: (i, 0))] * 2,
        out_shape=[jax.ShapeDtypeStruct((npad, hd2), jnp.float32)] * 2,
        compiler_params=pltpu.CompilerParams(
            dimension_semantics=("arbitrary",)),
    )


def _make_fin_kernel(npad, t_len, d_in, hid, hd2, dk, heads, blk):
    nblk = npad // blk
    hd = dk // heads
    scale = hd ** -0.5
    tdi = t_len * d_in

    def body(acc, accd, r_mat, m_mat, gb, x2b, wpbig, bpbig,
             wqT, bq, wkT, bk, wvT, bv, woT, bo,
             fw1T, fb1, fw2T, fb2, rw1T, rb1, rw2T, rb2,
             tout, fout, rout, xs_s, q_s, k_s, v_s, srow_s):
        xp2 = _gat_epilogue(acc, accd, r_mat, m_mat, gb)
        xp = jnp.transpose(xp2)
        xsf = jnp.dot(wpbig[...], x2b[...],
                      preferred_element_type=jnp.float32) + bpbig[...]
        xs_s[...] = xsf.reshape(t_len, hid, blk)

        def qkv(t, _):
            z = xs_s[t] + xp
            q_s[t] = jnp.dot(wqT[...], z,
                             preferred_element_type=jnp.float32) + bq[...]
            k_s[t] = jnp.dot(wkT[...], z,
                             preferred_element_type=jnp.float32) + bk[...]
            v_s[t] = jnp.dot(wvT[...], z,
                             preferred_element_type=jnp.float32) + bv[...]
            return 0

        lax.fori_loop(0, t_len, qkv, 0)

        def attend(t, _):
            qt = q_s[t] * scale

            def score(u, _):
                prod = (qt * k_s[u]).reshape(heads, hd, blk)
                srow_s[u] = jnp.sum(prod, axis=1)
                return 0

            lax.fori_loop(0, t_len, score, 0)
            s = srow_s[...]
            m = jnp.max(s, axis=0, keepdims=True)
            e = jnp.exp(s - m)
            srow_s[...] = e / jnp.sum(e, axis=0, keepdims=True)

            def accum(u, c):
                return c + srow_s[u][:, None, :] * v_s[u].reshape(
                    heads, hd, blk)

            ctx = lax.fori_loop(
                0, t_len, accum, jnp.zeros((heads, hd, blk), jnp.float32))
            ot = (jnp.dot(woT[...], ctx.reshape(dk, blk),
                          preferred_element_type=jnp.float32) + bo[...])
            tout[t] = jnp.transpose(ot)
            return 0

        lax.fori_loop(0, t_len, attend, 0)

        last = jnp.transpose(tout[t_len - 1])
        h1 = jax.nn.relu(jnp.dot(fw1T[...], last,
                                 preferred_element_type=jnp.float32)
                         + fb1[...])
        f = jax.nn.relu(jnp.dot(fw2T[...], h1,
                                preferred_element_type=jnp.float32)
                        + fb2[...])
        fout[...] = jnp.transpose(f)
        h2 = jax.nn.relu(jnp.dot(rw1T[...], last,
                                 preferred_element_type=jnp.float32)
                         + rb1[...])
        r = jax.nn.sigmoid(jnp.dot(rw2T[...], h2,
                                   preferred_element_type=jnp.float32)
                           + rb2[...])
        rout[...] = jnp.transpose(r)

    return pl.pallas_call(
        body,
        grid=(nblk,),
        in_specs=[
            pl.BlockSpec((2, blk, hd2), lambda i: (0, i, 0)),
            pl.BlockSpec((2, blk, L), lambda i: (0, i, 0)),
            _full((L, hd2)), _full((hd2, hid)), _full((1, hid)),
            pl.BlockSpec((tdi, blk), lambda i: (0, i)),
            _full((t_len * hid, tdi)), _full((t_len * hid, 1)),
            _full((dk, hid)), _full((dk, 1)),
            _full((dk, hid)), _full((dk, 1)),
            _full((dk, hid)), _full((dk, 1)),
            _full((hid, dk)), _full((hid, 1)),
            _full((hid, hid)), _full((hid, 1)),
            _full((1, hid)), _full((1, 1)),
            _full((hid, hid)), _full((hid, 1)),
            _full((1, hid)), _full((1, 1)),
        ],
        out_specs=[
            pl.BlockSpec((t_len, blk, hid), lambda i: (0, i, 0)),
            pl.BlockSpec((blk, 1), lambda i: (i, 0)),
            pl.BlockSpec((blk, 1), lambda i: (i, 0)),
        ],
        out_shape=[
            jax.ShapeDtypeStruct((t_len, npad, hid), jnp.float32),
            jax.ShapeDtypeStruct((npad, 1), jnp.float32),
            jax.ShapeDtypeStruct((npad, 1), jnp.float32),
        ],
        scratch_shapes=[
            pltpu.VMEM((t_len, hid, blk), jnp.float32),
            pltpu.VMEM((t_len, dk, blk), jnp.float32),
            pltpu.VMEM((t_len, dk, blk), jnp.float32),
            pltpu.VMEM((t_len, dk, blk), jnp.float32),
            pltpu.VMEM((t_len, heads, blk), jnp.float32),
        ],
        compiler_params=pltpu.CompilerParams(
            dimension_semantics=("arbitrary",)),
    )


def kernel(x, edge_index, edge_weight, Wp, bp, g0_Wl, g0_bl, g0_Wr, g0_br,
           g0_We, g0_att, g0_b, g1_Wl, g1_bl, g1_Wr, g1_br, g1_We, g1_att,
           g1_b, Wq, bq, Wk, bk, Wv, bv, Wo, bo, f_W1, f_b1, f_W2, f_b2,
           r_W1, r_b1, r_W2, r_b2):
    n, t_len, d_in = x.shape
    e = edge_weight.shape[0]
    hid = Wp.shape[1]
    dk = Wq.shape[1]
    heads, c = g0_att.shape
    hd2 = heads * c
    blk = 1024
    npad = -(-n // blk) * blk
    src = edge_index[0]
    dst = edge_index[1]

    epack = jnp.concatenate(
        [src[None], dst[None],
         jax.lax.bitcast_convert_type(edge_weight, jnp.int32)[None],
         jnp.zeros((1, e), jnp.int32)], axis=0)
    x2 = x.reshape(n, t_len * d_in)
    x2p = jnp.pad(x2, ((0, npad - n), (0, 0)))
    x2T = x2p.T
    r_mat = (jnp.arange(hd2)[None, :] // c == jnp.arange(L)[:, None]
             ).astype(jnp.float32)
    m_mat = (jnp.arange(hd2)[:, None] % c == jnp.arange(c)[None, :]
             ).astype(jnp.float32) / heads
    wpbig = jnp.kron(jnp.eye(t_len, dtype=jnp.float32), Wp.T)
    bpbig = jnp.tile(bp, t_len)[:, None]
    wl0 = Wp @ g0_Wl
    bl0 = (bp @ g0_Wl + g0_bl)[None]
    wr0 = Wp @ g0_Wr
    br0 = (bp @ g0_Wr + g0_br)[None]

    prep = _make_prep_kernel(npad, t_len, d_in, hd2, 2048)
    xl0, xr0 = prep(x2p, wl0, bl0, wr0, br0)

    ek = _make_edge_kernel(n, npad, e, hd2)
    acc0, accd0 = ek(xl0, xr0, epack, g0_We.reshape(hd2),
                     g0_att.reshape(hd2))

    mid = _make_mid_kernel(npad, hd2, hid, 2048)
    xl1, xr1 = mid(acc0.reshape(2, npad, hd2), accd0.reshape(2, npad, L),
                   r_mat, m_mat, g0_b[None],
                   g1_Wl, g1_bl[None], g1_Wr, g1_br[None])

    acc1, accd1 = ek(xl1, xr1, epack, g1_We.reshape(hd2),
                     g1_att.reshape(hd2))

    fin = _make_fin_kernel(npad, t_len, d_in, hid, hd2, dk, heads, blk)
    tout, fout, rout = fin(
        acc1.reshape(2, npad, hd2), accd1.reshape(2, npad, L),
        r_mat, m_mat, g1_b[None], x2T, wpbig, bpbig,
        Wq.T, bq[:, None], Wk.T, bk[:, None], Wv.T, bv[:, None],
        Wo.T, bo[:, None], f_W1.T, f_b1[:, None], f_W2.T, f_b2[:, None],
        r_W1.T, r_b1[:, None], r_W2.T, r_b2[:, None])
    return (fout[:n], rout[:n], tout[:, :n])
